# baseline (speedup 1.0000x reference)
import jax
import jax.numpy as jnp
from jax.experimental import pallas as pl
from jax.experimental.pallas import tpu as pltpu

_EMBED = 32
_K = 1024
_N = 16384
_BLOCK = 2048
_SL = 128
_SUB = 512
_COMMIT = 0.25


def _split3(emb):
    hi = emb.astype(jnp.bfloat16).astype(jnp.float32)
    r = emb - hi
    mid = r.astype(jnp.bfloat16).astype(jnp.float32)
    lo = r - mid
    return jnp.concatenate([hi, mid, lo], axis=1)


def _stage(x, emb2x, e2, e3, enc_ref, row0):
    ns = _K // _SL
    x2 = jnp.sum(x ** 2, axis=1, keepdims=True)
    mm2 = jax.lax.dot_general(x, emb2x, (((1,), (1,)), ((), ())))

    mval = None
    midx = None
    for s in range(ns):
        lo = s * _SL
        iota_s = jax.lax.broadcasted_iota(jnp.int32, (x.shape[0], _SL), 1) + lo
        ds = (x2 + e2[:, lo:lo + _SL]) - mm2[:, lo:lo + _SL]
        if s == 0:
            mval, midx = ds, iota_s
        else:
            pred = ds < mval
            midx = jnp.where(pred, iota_s, midx)
            mval = jnp.minimum(mval, ds)
    m = jnp.min(mval, axis=1, keepdims=True)
    idx = jnp.min(jnp.where(mval == m, midx, _K), axis=1, keepdims=True)

    col = jax.lax.broadcasted_iota(jnp.int32, (x.shape[0], _K), 1)
    enc = jnp.where(col == idx, 1.0, 0.0).astype(jnp.float32)
    enc_ref[row0:row0 + _SUB, :] = enc
    q3 = jax.lax.dot(enc, e3)
    q = (q3[:, 0:_EMBED] + q3[:, _EMBED:2 * _EMBED]) + q3[:, 2 * _EMBED:]
    return enc, q


def _vq_kernel(x_ref, e0_ref, e1_ref,
               q_ref, enc0_ref, enc1_ref, stats_ref,
               cnt_ref, acc_ref, e2_ref, e3_ref, e2x_ref):
    i = pl.program_id(0)
    nsteps = pl.num_programs(0)

    @pl.when(i == 0)
    def _init():
        cnt_ref[...] = jnp.zeros_like(cnt_ref)
        acc_ref[...] = jnp.zeros_like(acc_ref)
        e0 = e0_ref[...]
        e1 = e1_ref[...]
        e2_ref[0:1, :] = jnp.sum(e0 ** 2, axis=1)[None, :]
        e2_ref[1:2, :] = jnp.sum(e1 ** 2, axis=1)[None, :]
        e3_ref[0:_K, :] = _split3(e0)
        e3_ref[_K:, :] = _split3(e1)
        e2x_ref[0:_K, :] = e0 + e0
        e2x_ref[_K:, :] = e1 + e1

    ones = jnp.ones((8, _SUB), jnp.float32)
    for j in range(_BLOCK // _SUB):
        r0, r1o = j * _SUB, (j + 1) * _SUB
        x = x_ref[r0:r1o, :]
        enc0, q0 = _stage(x, e2x_ref[0:_K, :], e2_ref[0:1, :],
                          e3_ref[0:_K, :], enc0_ref, r0)
        r1 = x - q0
        enc1, q1 = _stage(r1, e2x_ref[_K:, :], e2_ref[1:2, :],
                          e3_ref[_K:, :], enc1_ref, r0)

        q_ref[r0:r1o, :] = q0 + q1

        cnt_ref[0:8, :] += jax.lax.dot(ones, enc0)
        cnt_ref[8:16, :] += jax.lax.dot(ones, enc1)

        ssq = jnp.sum((q0 - x) ** 2) + jnp.sum((q1 - r1) ** 2)
        acc_ref[...] += ssq

    @pl.when(i == nsteps - 1)
    def _finish():
        loss_tile = acc_ref[...] * ((1.0 + _COMMIT) / (_N * _EMBED))
        p = cnt_ref[...] / _N
        s = jnp.sum(p * jnp.log(p + 1e-10), axis=1)
        ent = jnp.exp(-s)
        rid = jax.lax.broadcasted_iota(jnp.int32, ent.shape, 0)
        ent_tot = jnp.sum(jnp.where((rid == 0) | (rid == 8), ent, 0.0))
        row = jax.lax.broadcasted_iota(jnp.int32, stats_ref.shape, 0)
        colm = jax.lax.broadcasted_iota(jnp.int32, stats_ref.shape, 1)
        stats_ref[...] = jnp.where((row == 0) & (colm == 0), loss_tile,
                                   jnp.where((row == 0) & (colm == 1),
                                             ent_tot, 0.0))


def _run(flat, e0, e1, interpret=False):
    grid = (_N // _BLOCK,)
    q, enc0, enc1, stats = pl.pallas_call(
        _vq_kernel,
        grid=grid,
        in_specs=[
            pl.BlockSpec((_BLOCK, _EMBED), lambda i: (i, 0)),
            pl.BlockSpec((_K, _EMBED), lambda i: (0, 0)),
            pl.BlockSpec((_K, _EMBED), lambda i: (0, 0)),
        ],
        out_specs=[
            pl.BlockSpec((_BLOCK, _EMBED), lambda i: (i, 0)),
            pl.BlockSpec((_BLOCK, _K), lambda i: (i, 0)),
            pl.BlockSpec((_BLOCK, _K), lambda i: (i, 0)),
            pl.BlockSpec((8, 128), lambda i: (0, 0)),
        ],
        out_shape=[
            jax.ShapeDtypeStruct((_N, _EMBED), jnp.float32),
            jax.ShapeDtypeStruct((_N, _K), jnp.float32),
            jax.ShapeDtypeStruct((_N, _K), jnp.float32),
            jax.ShapeDtypeStruct((8, 128), jnp.float32),
        ],
        scratch_shapes=[
            pltpu.VMEM((16, _K), jnp.float32),
            pltpu.VMEM((8, 128), jnp.float32),
            pltpu.VMEM((8, _K), jnp.float32),
            pltpu.VMEM((2 * _K, 3 * _EMBED), jnp.float32),
            pltpu.VMEM((2 * _K, _EMBED), jnp.float32),
        ],
        interpret=interpret,
    )(flat, e0, e1)
    return q, enc0, enc1, stats


def kernel(data, codebook0, codebook1):
    flat = data.reshape(-1, _EMBED)
    q, enc0, enc1, stats = _run(flat, codebook0, codebook1)
    quantized = q.reshape(data.shape)
    loss = stats[0, 0]
    entropy = stats[0, 1]
    return (quantized, (enc0, enc1), loss, entropy)


# speedup vs baseline: 1.0711x; 1.0711x over previous
import jax
import jax.numpy as jnp
from jax.experimental import pallas as pl
from jax.experimental.pallas import tpu as pltpu

_EMBED = 32
_K = 1024
_N = 16384
_BLOCK = 2048
_SL = 128
_SUB = 1024
_COMMIT = 0.25


def _split3(emb):
    hi = emb.astype(jnp.bfloat16).astype(jnp.float32)
    r = emb - hi
    mid = r.astype(jnp.bfloat16).astype(jnp.float32)
    lo = r - mid
    return jnp.concatenate([hi, mid, lo], axis=1)


def _stage(x, emb2x, e2, e3, enc_ref, row0):
    ns = _K // _SL
    x2 = jnp.sum(x ** 2, axis=1, keepdims=True)
    mm2 = jax.lax.dot_general(x, emb2x, (((1,), (1,)), ((), ())))

    mval = None
    midx = None
    for s in range(ns):
        lo = s * _SL
        iota_s = jax.lax.broadcasted_iota(jnp.int32, (x.shape[0], _SL), 1) + lo
        ds = (x2 + e2[:, lo:lo + _SL]) - mm2[:, lo:lo + _SL]
        if s == 0:
            mval, midx = ds, iota_s
        else:
            pred = ds < mval
            midx = jnp.where(pred, iota_s, midx)
            mval = jnp.minimum(mval, ds)
    m = jnp.min(mval, axis=1, keepdims=True)
    idx = jnp.min(jnp.where(mval == m, midx, _K), axis=1, keepdims=True)

    col = jax.lax.broadcasted_iota(jnp.int32, (x.shape[0], _K), 1)
    enc = jnp.where(col == idx, 1.0, 0.0).astype(jnp.float32)
    enc_ref[row0:row0 + _SUB, :] = enc
    q3 = jax.lax.dot(enc, e3)
    q = (q3[:, 0:_EMBED] + q3[:, _EMBED:2 * _EMBED]) + q3[:, 2 * _EMBED:]
    return enc, q


def _vq_kernel(x_ref, e0_ref, e1_ref,
               q_ref, enc0_ref, enc1_ref, stats_ref,
               cnt_ref, acc_ref, e2_ref, e3_ref, e2x_ref):
    i = pl.program_id(0)
    nsteps = pl.num_programs(0)

    @pl.when(i == 0)
    def _init():
        cnt_ref[...] = jnp.zeros_like(cnt_ref)
        acc_ref[...] = jnp.zeros_like(acc_ref)
        e0 = e0_ref[...]
        e1 = e1_ref[...]
        e2_ref[0:1, :] = jnp.sum(e0 ** 2, axis=1)[None, :]
        e2_ref[1:2, :] = jnp.sum(e1 ** 2, axis=1)[None, :]
        e3_ref[0:_K, :] = _split3(e0)
        e3_ref[_K:, :] = _split3(e1)
        e2x_ref[0:_K, :] = e0 + e0
        e2x_ref[_K:, :] = e1 + e1

    ones = jnp.ones((8, _SUB), jnp.float32)
    for j in range(_BLOCK // _SUB):
        r0, r1o = j * _SUB, (j + 1) * _SUB
        x = x_ref[r0:r1o, :]
        enc0, q0 = _stage(x, e2x_ref[0:_K, :], e2_ref[0:1, :],
                          e3_ref[0:_K, :], enc0_ref, r0)
        r1 = x - q0
        enc1, q1 = _stage(r1, e2x_ref[_K:, :], e2_ref[1:2, :],
                          e3_ref[_K:, :], enc1_ref, r0)

        q_ref[r0:r1o, :] = q0 + q1

        cnt_ref[0:8, :] += jax.lax.dot(ones, enc0)
        cnt_ref[8:16, :] += jax.lax.dot(ones, enc1)

        ssq = jnp.sum((q0 - x) ** 2) + jnp.sum((q1 - r1) ** 2)
        acc_ref[...] += ssq

    @pl.when(i == nsteps - 1)
    def _finish():
        loss_tile = acc_ref[...] * ((1.0 + _COMMIT) / (_N * _EMBED))
        p = cnt_ref[...] / _N
        s = jnp.sum(p * jnp.log(p + 1e-10), axis=1)
        ent = jnp.exp(-s)
        rid = jax.lax.broadcasted_iota(jnp.int32, ent.shape, 0)
        ent_tot = jnp.sum(jnp.where((rid == 0) | (rid == 8), ent, 0.0))
        row = jax.lax.broadcasted_iota(jnp.int32, stats_ref.shape, 0)
        colm = jax.lax.broadcasted_iota(jnp.int32, stats_ref.shape, 1)
        stats_ref[...] = jnp.where((row == 0) & (colm == 0), loss_tile,
                                   jnp.where((row == 0) & (colm == 1),
                                             ent_tot, 0.0))


def _run(flat, e0, e1, interpret=False):
    grid = (_N // _BLOCK,)
    q, enc0, enc1, stats = pl.pallas_call(
        _vq_kernel,
        grid=grid,
        in_specs=[
            pl.BlockSpec((_BLOCK, _EMBED), lambda i: (i, 0)),
            pl.BlockSpec((_K, _EMBED), lambda i: (0, 0)),
            pl.BlockSpec((_K, _EMBED), lambda i: (0, 0)),
        ],
        out_specs=[
            pl.BlockSpec((_BLOCK, _EMBED), lambda i: (i, 0)),
            pl.BlockSpec((_BLOCK, _K), lambda i: (i, 0)),
            pl.BlockSpec((_BLOCK, _K), lambda i: (i, 0)),
            pl.BlockSpec((8, 128), lambda i: (0, 0)),
        ],
        out_shape=[
            jax.ShapeDtypeStruct((_N, _EMBED), jnp.float32),
            jax.ShapeDtypeStruct((_N, _K), jnp.float32),
            jax.ShapeDtypeStruct((_N, _K), jnp.float32),
            jax.ShapeDtypeStruct((8, 128), jnp.float32),
        ],
        scratch_shapes=[
            pltpu.VMEM((16, _K), jnp.float32),
            pltpu.VMEM((8, 128), jnp.float32),
            pltpu.VMEM((8, _K), jnp.float32),
            pltpu.VMEM((2 * _K, 3 * _EMBED), jnp.float32),
            pltpu.VMEM((2 * _K, _EMBED), jnp.float32),
        ],
        interpret=interpret,
    )(flat, e0, e1)
    return q, enc0, enc1, stats


def kernel(data, codebook0, codebook1):
    flat = data.reshape(-1, _EMBED)
    q, enc0, enc1, stats = _run(flat, codebook0, codebook1)
    quantized = q.reshape(data.shape)
    loss = stats[0, 0]
    entropy = stats[0, 1]
    return (quantized, (enc0, enc1), loss, entropy)


# speedup vs baseline: 1.1219x; 1.0475x over previous
import jax
import jax.numpy as jnp
from jax.experimental import pallas as pl
from jax.experimental.pallas import tpu as pltpu

_EMBED = 32
_K = 1024
_N = 16384
_BLOCK = 2048
_SL = 128
_SUB = 2048
_COMMIT = 0.25


def _split3(emb):
    hi = emb.astype(jnp.bfloat16).astype(jnp.float32)
    r = emb - hi
    mid = r.astype(jnp.bfloat16).astype(jnp.float32)
    lo = r - mid
    return jnp.concatenate([hi, mid, lo], axis=1)


def _stage(x, emb2x, e2, e3, enc_ref, row0):
    ns = _K // _SL
    x2 = jnp.sum(x ** 2, axis=1, keepdims=True)
    mm2 = jax.lax.dot_general(x, emb2x, (((1,), (1,)), ((), ())))

    mval = None
    midx = None
    for s in range(ns):
        lo = s * _SL
        iota_s = jax.lax.broadcasted_iota(jnp.int32, (x.shape[0], _SL), 1) + lo
        ds = (x2 + e2[:, lo:lo + _SL]) - mm2[:, lo:lo + _SL]
        if s == 0:
            mval, midx = ds, iota_s
        else:
            pred = ds < mval
            midx = jnp.where(pred, iota_s, midx)
            mval = jnp.minimum(mval, ds)
    m = jnp.min(mval, axis=1, keepdims=True)
    idx = jnp.min(jnp.where(mval == m, midx, _K), axis=1, keepdims=True)

    col = jax.lax.broadcasted_iota(jnp.int32, (x.shape[0], _K), 1)
    enc = jnp.where(col == idx, 1.0, 0.0).astype(jnp.float32)
    enc_ref[row0:row0 + _SUB, :] = enc
    q3 = jax.lax.dot(enc, e3)
    q = (q3[:, 0:_EMBED] + q3[:, _EMBED:2 * _EMBED]) + q3[:, 2 * _EMBED:]
    return enc, q


def _vq_kernel(x_ref, e0_ref, e1_ref,
               q_ref, enc0_ref, enc1_ref, stats_ref,
               cnt_ref, acc_ref, e2_ref, e3_ref, e2x_ref):
    i = pl.program_id(0)
    nsteps = pl.num_programs(0)

    @pl.when(i == 0)
    def _init():
        cnt_ref[...] = jnp.zeros_like(cnt_ref)
        acc_ref[...] = jnp.zeros_like(acc_ref)
        e0 = e0_ref[...]
        e1 = e1_ref[...]
        e2_ref[0:1, :] = jnp.sum(e0 ** 2, axis=1)[None, :]
        e2_ref[1:2, :] = jnp.sum(e1 ** 2, axis=1)[None, :]
        e3_ref[0:_K, :] = _split3(e0)
        e3_ref[_K:, :] = _split3(e1)
        e2x_ref[0:_K, :] = e0 + e0
        e2x_ref[_K:, :] = e1 + e1

    ones = jnp.ones((8, _SUB), jnp.float32)
    for j in range(_BLOCK // _SUB):
        r0, r1o = j * _SUB, (j + 1) * _SUB
        x = x_ref[r0:r1o, :]
        enc0, q0 = _stage(x, e2x_ref[0:_K, :], e2_ref[0:1, :],
                          e3_ref[0:_K, :], enc0_ref, r0)
        r1 = x - q0
        enc1, q1 = _stage(r1, e2x_ref[_K:, :], e2_ref[1:2, :],
                          e3_ref[_K:, :], enc1_ref, r0)

        q_ref[r0:r1o, :] = q0 + q1

        cnt_ref[0:8, :] += jax.lax.dot(ones, enc0)
        cnt_ref[8:16, :] += jax.lax.dot(ones, enc1)

        ssq = jnp.sum((q0 - x) ** 2) + jnp.sum((q1 - r1) ** 2)
        acc_ref[...] += ssq

    @pl.when(i == nsteps - 1)
    def _finish():
        loss_tile = acc_ref[...] * ((1.0 + _COMMIT) / (_N * _EMBED))
        p = cnt_ref[...] / _N
        s = jnp.sum(p * jnp.log(p + 1e-10), axis=1)
        ent = jnp.exp(-s)
        rid = jax.lax.broadcasted_iota(jnp.int32, ent.shape, 0)
        ent_tot = jnp.sum(jnp.where((rid == 0) | (rid == 8), ent, 0.0))
        row = jax.lax.broadcasted_iota(jnp.int32, stats_ref.shape, 0)
        colm = jax.lax.broadcasted_iota(jnp.int32, stats_ref.shape, 1)
        stats_ref[...] = jnp.where((row == 0) & (colm == 0), loss_tile,
                                   jnp.where((row == 0) & (colm == 1),
                                             ent_tot, 0.0))


def _run(flat, e0, e1, interpret=False):
    grid = (_N // _BLOCK,)
    q, enc0, enc1, stats = pl.pallas_call(
        _vq_kernel,
        grid=grid,
        in_specs=[
            pl.BlockSpec((_BLOCK, _EMBED), lambda i: (i, 0)),
            pl.BlockSpec((_K, _EMBED), lambda i: (0, 0)),
            pl.BlockSpec((_K, _EMBED), lambda i: (0, 0)),
        ],
        out_specs=[
            pl.BlockSpec((_BLOCK, _EMBED), lambda i: (i, 0)),
            pl.BlockSpec((_BLOCK, _K), lambda i: (i, 0)),
            pl.BlockSpec((_BLOCK, _K), lambda i: (i, 0)),
            pl.BlockSpec((8, 128), lambda i: (0, 0)),
        ],
        out_shape=[
            jax.ShapeDtypeStruct((_N, _EMBED), jnp.float32),
            jax.ShapeDtypeStruct((_N, _K), jnp.float32),
            jax.ShapeDtypeStruct((_N, _K), jnp.float32),
            jax.ShapeDtypeStruct((8, 128), jnp.float32),
        ],
        scratch_shapes=[
            pltpu.VMEM((16, _K), jnp.float32),
            pltpu.VMEM((8, 128), jnp.float32),
            pltpu.VMEM((8, _K), jnp.float32),
            pltpu.VMEM((2 * _K, 3 * _EMBED), jnp.float32),
            pltpu.VMEM((2 * _K, _EMBED), jnp.float32),
        ],
        interpret=interpret,
    )(flat, e0, e1)
    return q, enc0, enc1, stats


def kernel(data, codebook0, codebook1):
    flat = data.reshape(-1, _EMBED)
    q, enc0, enc1, stats = _run(flat, codebook0, codebook1)
    quantized = q.reshape(data.shape)
    loss = stats[0, 0]
    entropy = stats[0, 1]
    return (quantized, (enc0, enc1), loss, entropy)
